# baseline (speedup 1.0000x reference)
import functools

import jax
import jax.numpy as jnp
from jax import lax
from jax.experimental import pallas as pl
from jax.experimental.pallas import tpu as pltpu
from jax.experimental.pallas import tpu_sc as plsc

_B, _T, _D = 4, 2048, 768
_DP = _D + 128
_N = _B * _T
_E = 8
_EP = 128
_BM = 512
_NBLK = _N // _BM
_BMC = 256
_NS = _N * 2 + _E * _BMC
_NT = _NS // _BMC
_NEG = -1e30

_NW = 32
_TPW = _N // _NW
_CH = 64
_NCH = _TPW // _CH


def _route_block(x_ref, gw_ref, gb_ref, xw1_ref, xw2_ref, meta_ref, pg_ref,
                 counts_ref, carry_ref):
    i = pl.program_id(0)

    @pl.when(i == 0)
    def _():
        carry_ref[...] = jnp.zeros((1, _E), jnp.float32)

    x = x_ref[...]
    logits = lax.dot_general(x, gw_ref[...], (((1,), (1,)), ((), ())),
                             preferred_element_type=jnp.float32) + gb_ref[...]
    eidx = lax.broadcasted_iota(jnp.int32, logits.shape, 1)
    m1 = jnp.max(logits, axis=1, keepdims=True)
    a1 = jnp.min(jnp.where(logits >= m1, eidx, _EP), axis=1, keepdims=True)
    mask1 = eidx == a1
    l2 = jnp.where(mask1, _NEG, logits)
    m2 = jnp.max(l2, axis=1, keepdims=True)
    a2 = jnp.min(jnp.where(l2 >= m2, eidx, _EP), axis=1, keepdims=True)
    mask2 = eidx == a2
    w1 = jax.nn.sigmoid(m1)
    w2 = jax.nn.sigmoid(m2)

    z = jnp.zeros((_BM, _DP - _D - 1), jnp.float32)
    xw1_ref[...] = jnp.concatenate([w1 * x, w1, z], axis=1)
    xw2_ref[...] = jnp.concatenate([w2 * x, w2, z], axis=1)

    m1e = mask1[:, :_E]
    m2e = mask2[:, :_E]
    cnt = m1e.astype(jnp.float32) + m2e.astype(jnp.float32)
    r_io = lax.broadcasted_iota(jnp.int32, (_BM, _BM), 0)
    c_io = lax.broadcasted_iota(jnp.int32, (_BM, _BM), 1)
    ltri = (r_io > c_io).astype(jnp.float32)
    excl = lax.dot_general(ltri, cnt, (((1,), (0,)), ((), ())),
                           preferred_element_type=jnp.float32)
    rankf = excl + carry_ref[...]
    r1 = jnp.sum(jnp.where(m1e, rankf, 0.0), axis=1, keepdims=True)
    r2 = jnp.sum(jnp.where(m2e, rankf, 0.0), axis=1, keepdims=True)
    carry_ref[...] = carry_ref[...] + jnp.sum(cnt, axis=0, keepdims=True)

    zi = jnp.zeros((_BM, 2), jnp.int32)
    meta_ref[...] = jnp.concatenate(
        [a1, a2, r1.astype(jnp.int32), r2.astype(jnp.int32), zi, zi], axis=1)

    @pl.when(i == _NBLK - 1)
    def _():
        total = carry_ref[...]
        aligned = jnp.ceil(total / _BMC) * _BMC
        e_r = lax.broadcasted_iota(jnp.int32, (_E, _E), 0)
        e_c = lax.broadcasted_iota(jnp.int32, (_E, _E), 1)
        before = (e_r < e_c).astype(jnp.float32)
        pg = lax.dot_general(aligned, before, (((1,), (0,)), ((), ())),
                             preferred_element_type=jnp.float32)
        pg_ref[...] = jnp.concatenate(
            [pg.astype(jnp.int32), jnp.zeros((1, 16 - _E), jnp.int32)],
            axis=1)
        counts_ref[...] = total.astype(jnp.int32)


def _route(x, gw_p, gb_p):
    return pl.pallas_call(
        _route_block,
        grid=(_NBLK,),
        in_specs=[
            pl.BlockSpec((_BM, _D), lambda i: (i, 0)),
            pl.BlockSpec((_EP, _D), lambda i: (0, 0)),
            pl.BlockSpec((1, _EP), lambda i: (0, 0)),
        ],
        out_specs=[
            pl.BlockSpec((_BM, _DP), lambda i: (i, 0)),
            pl.BlockSpec((_BM, _DP), lambda i: (i, 0)),
            pl.BlockSpec((_BM, 8), lambda i: (i, 0)),
            pl.BlockSpec((1, 16), lambda i: (0, 0)),
            pl.BlockSpec((1, _E), lambda i: (0, 0)),
        ],
        out_shape=[
            jax.ShapeDtypeStruct((_N, _DP), jnp.float32),
            jax.ShapeDtypeStruct((_N, _DP), jnp.float32),
            jax.ShapeDtypeStruct((_N, 8), jnp.int32),
            jax.ShapeDtypeStruct((1, 16), jnp.int32),
            jax.ShapeDtypeStruct((1, _E), jnp.int32),
        ],
        scratch_shapes=[pltpu.VMEM((1, _E), jnp.float32)],
        compiler_params=pltpu.CompilerParams(
            dimension_semantics=("arbitrary",),
        ),
    )(x, gw_p, gb_p)


def _pos_block(meta_ref, pg_ref, pos1_ref, pos2_ref):
    m = meta_ref[...]
    pg = pg_ref[...]
    ei = lax.broadcasted_iota(jnp.int32, (_BM, 16), 1)
    s1 = jnp.sum(jnp.where(ei == m[:, 0:1], pg, 0), axis=1)
    s2 = jnp.sum(jnp.where(ei == m[:, 1:2], pg, 0), axis=1)
    pos1_ref[...] = s1 + m[:, 2]
    pos2_ref[...] = s2 + m[:, 3]


def _positions(meta, pg16):
    return pl.pallas_call(
        _pos_block,
        grid=(_NBLK,),
        in_specs=[
            pl.BlockSpec((_BM, 8), lambda i: (i, 0)),
            pl.BlockSpec((1, 16), lambda i: (0, 0)),
        ],
        out_specs=[
            pl.BlockSpec((_BM,), lambda i: (i,)),
            pl.BlockSpec((_BM,), lambda i: (i,)),
        ],
        out_shape=[
            jax.ShapeDtypeStruct((_N,), jnp.int32),
            jax.ShapeDtypeStruct((_N,), jnp.int32),
        ],
        compiler_params=pltpu.CompilerParams(
            dimension_semantics=("arbitrary",),
        ),
    )(meta, pg16)


def _sc_wid():
    return lax.axis_index("s") * 2 + lax.axis_index("c")


@functools.cache
def _build_dispatch():
    mesh = plsc.VectorSubcoreMesh(core_axis_name="c", subcore_axis_name="s")

    @functools.partial(
        pl.kernel, mesh=mesh,
        out_type=jax.ShapeDtypeStruct((_NS, _DP), jnp.float32),
        scratch_types=[
            pltpu.VMEM((_CH,), jnp.int32),
            pltpu.VMEM((_CH,), jnp.int32),
            pltpu.VMEM((_CH, _DP), jnp.float32),
            pltpu.VMEM((_CH, _DP), jnp.float32),
            pltpu.SemaphoreType.DMA,
            pltpu.SemaphoreType.DMA,
        ],
    )
    def _dispatch(xw1_hbm, xw2_hbm, pos1_hbm, pos2_hbm, xs_hbm,
                  p1, p2, v1, v2, sem1, sem2):
        base = _sc_wid() * _TPW
        for ci in range(_NCH):
            t0 = base + ci * _CH
            pltpu.sync_copy(pos1_hbm.at[pl.ds(t0, _CH)], p1)
            pltpu.sync_copy(pos2_hbm.at[pl.ds(t0, _CH)], p2)
            pltpu.sync_copy(xw1_hbm.at[pl.ds(t0, _CH)], v1)
            pltpu.sync_copy(xw2_hbm.at[pl.ds(t0, _CH)], v2)
            cp1 = pltpu.async_copy(v1, xs_hbm.at[p1], sem1)
            cp2 = pltpu.async_copy(v2, xs_hbm.at[p2], sem2)
            cp1.wait()
            cp2.wait()

    return _dispatch


def _gmm_block(te_ref, xs_ref, w_ref, ys_ref):
    xb = xs_ref[...].astype(jnp.bfloat16)
    ys_ref[...] = lax.dot_general(xb, w_ref[0], (((1,), (1,)), ((), ())),
                                  preferred_element_type=jnp.float32)


def _gmm(xs, wp_bf16, tile_expert):
    return pl.pallas_call(
        _gmm_block,
        grid_spec=pltpu.PrefetchScalarGridSpec(
            num_scalar_prefetch=1,
            grid=(_NT,),
            in_specs=[
                pl.BlockSpec((_BMC, _DP), lambda i, te: (i, 0)),
                pl.BlockSpec((1, _D, _DP), lambda i, te: (te[i], 0, 0)),
            ],
            out_specs=pl.BlockSpec((_BMC, _D), lambda i, te: (i, 0)),
        ),
        out_shape=jax.ShapeDtypeStruct((_NS, _D), jnp.float32),
        compiler_params=pltpu.CompilerParams(
            dimension_semantics=("arbitrary",),
        ),
    )(tile_expert, xs, wp_bf16)


@functools.cache
def _build_combine():
    mesh = plsc.VectorSubcoreMesh(core_axis_name="c", subcore_axis_name="s")

    @functools.partial(
        pl.kernel, mesh=mesh,
        out_type=jax.ShapeDtypeStruct((_N, _D), jnp.float32),
        scratch_types=[
            pltpu.VMEM((_CH,), jnp.int32),
            pltpu.VMEM((_CH,), jnp.int32),
            pltpu.VMEM((_CH, _D), jnp.float32),
            pltpu.VMEM((_CH, _D), jnp.float32),
            pltpu.SemaphoreType.DMA,
            pltpu.SemaphoreType.DMA,
        ],
    )
    def _combine(ys_hbm, pos1_hbm, pos2_hbm, y_hbm, p1, p2, b1, b2,
                 sem1, sem2):
        base = _sc_wid() * _TPW
        for ci in range(_NCH):
            t0 = base + ci * _CH
            pltpu.sync_copy(pos1_hbm.at[pl.ds(t0, _CH)], p1)
            pltpu.sync_copy(pos2_hbm.at[pl.ds(t0, _CH)], p2)
            cp1 = pltpu.async_copy(ys_hbm.at[p1], b1, sem1)
            cp2 = pltpu.async_copy(ys_hbm.at[p2], b2, sem2)
            cp1.wait()
            cp2.wait()

            def loop(t, _):
                for v in range(_D // 16):
                    sl = pl.ds(v * 16, 16)
                    b1[t, sl] = b1[t, sl] + b2[t, sl]
                return 0

            lax.fori_loop(0, _CH, loop, 0)
            pltpu.sync_copy(b1, y_hbm.at[pl.ds(t0, _CH)])

    return _combine


def kernel(inputs, gate_W, gate_b, expert_W, expert_b):
    x = inputs.reshape(_N, _D)
    gw_p = jnp.zeros((_EP, _D), jnp.float32).at[:_E].set(gate_W)
    gb_p = jnp.full((1, _EP), _NEG, jnp.float32).at[0, :_E].set(gate_b)
    wp = jnp.zeros((_E, _D, _DP), jnp.float32)
    wp = wp.at[:, :, :_D].set(expert_W).at[:, :, _D].set(expert_b)
    wp_bf16 = wp.astype(jnp.bfloat16)

    xw1, xw2, meta, pg16, counts = _route(x, gw_p, gb_p)
    pos1, pos2 = _positions(meta, pg16)

    aligned = (counts[0] + (_BMC - 1)) // _BMC * _BMC
    ends = jnp.cumsum(aligned)
    tile_starts = jnp.arange(_NT, dtype=jnp.int32) * _BMC
    tile_expert = jnp.sum(
        (tile_starts[:, None] >= ends[None, :_E - 1]).astype(jnp.int32),
        axis=1)

    xs = _build_dispatch()(xw1, xw2, pos1, pos2)
    ys = _gmm(xs, wp_bf16, tile_expert)
    y = _build_combine()(ys, pos1, pos2)
    return y.reshape(_B, _T, _D)
